# baseline (speedup 1.0000x reference)
import dataclasses
import functools

import jax
import jax.numpy as jnp
from jax import lax
from jax.experimental import pallas as pl
from jax.experimental.pallas import tpu as pltpu
from jax.experimental.pallas import tpu_sc as plsc

N = 10000
F = 128
ED = 16
E = 320000
ELG = 640000
G = 256
T = 3
S = 6 * F

NC = 2
NS = 16
CH = 12800
NCHUNK = E // CH
PER_SUB = ELG // NS
IB = 2000
NBLK = PER_SUB // IB
NVEC = IB // 16
BB = 128
CAP = BB + 16


def _msg_pass_kernel(out_hbm, lg0_hbm, lg1_hbm, ea_hbm, o_hbm,
                     acc, l0v, l1v, sbuf, dbuf, fsrc, fdst, rows, gsem):
    cid = lax.axis_index("c")
    sid = lax.axis_index("s")
    z16 = jnp.zeros((16,), jnp.int32)
    ch16 = jnp.full((16,), CH, jnp.int32)

    def fill_scrap(lo):
        for k in range(lo, CAP // 16):
            sbuf[pl.ds(k * 16, 16)] = z16
            dbuf[pl.ds(k * 16, 16)] = ch16

    def flush():
        for k in range(BB // 16):
            sv = sbuf[pl.ds(k * 16, 16)]
            dv = dbuf[pl.ds(k * 16, 16)]
            fsrc[0, pl.ds(k * 16, 16)] = jnp.minimum(
                jnp.maximum(sv, 0), E - 1)
            fdst[0, pl.ds(k * 16, 16)] = jnp.minimum(
                jnp.maximum(dv, 0), CH)
        pltpu.async_copy(out_hbm.at[fsrc.at[0]], rows, gsem).wait()
        pltpu.sync_copy(rows, acc.at[fdst.at[0]], add=True)

    nck = (NCHUNK + NC - 1) // NC

    @pl.loop(0, nck)
    def _chunk_loop(kc):
        chunk = cid * nck + kc

        @pl.when(chunk < NCHUNK)
        def _(chunk=chunk):
            base = chunk * CH
            row0 = base + sid * (CH // NS)
            pltpu.sync_copy(ea_hbm.at[pl.ds(row0, CH // NS)],
                            acc.at[pl.ds(sid * (CH // NS), CH // NS)])
            plsc.subcore_barrier()
            fill_scrap(0)

            def scan_vec(v, pos, base=base):
                d = l1v[pl.ds(v * 16, 16)]
                s = l0v[pl.ds(v * 16, 16)]
                dl = d - base
                m = (dl >= 0) & (dl < CH)
                plsc.store_compressed(dbuf.at[pl.ds(pos, 16)], dl, mask=m)
                plsc.store_compressed(sbuf.at[pl.ds(pos, 16)], s, mask=m)
                pos = pos + jnp.sum(m.astype(jnp.int32))
                flushed = pos >= BB

                @pl.when(flushed)
                def _():
                    flush()
                    rs = sbuf[pl.ds(BB, 16)]
                    rd = dbuf[pl.ds(BB, 16)]
                    sbuf[pl.ds(0, 16)] = rs
                    dbuf[pl.ds(0, 16)] = rd
                    fill_scrap(1)

                return jnp.where(flushed, pos - BB, pos)

            def blk_body(b, pos):
                off = sid * PER_SUB + b * IB
                pltpu.sync_copy(lg0_hbm.at[pl.ds(off, IB)], l0v)
                pltpu.sync_copy(lg1_hbm.at[pl.ds(off, IB)], l1v)
                return lax.fori_loop(0, NVEC, scan_vec, pos)

            lax.fori_loop(0, NBLK, blk_body, jnp.int32(0))
            flush()
            plsc.subcore_barrier()
            pltpu.sync_copy(acc.at[pl.ds(sid * (CH // NS), CH // NS)],
                            o_hbm.at[pl.ds(row0, CH // NS)])
            plsc.subcore_barrier()


def _msg_pass(out_old, lg0, lg1, ea):
    k = pl.kernel(
        _msg_pass_kernel,
        out_type=jax.ShapeDtypeStruct((E, F), jnp.float32),
        mesh=plsc.VectorSubcoreMesh(core_axis_name="c", subcore_axis_name="s"),
        scratch_types=[
            pltpu.VMEM_SHARED((CH + 8, F), jnp.float32),
            pltpu.VMEM((IB,), jnp.int32),
            pltpu.VMEM((IB,), jnp.int32),
            pltpu.VMEM((CAP,), jnp.int32),
            pltpu.VMEM((CAP,), jnp.int32),
            pltpu.VMEM((1, BB), jnp.int32),
            pltpu.VMEM((1, BB), jnp.int32),
            pltpu.VMEM((BB, F), jnp.float32),
            pltpu.SemaphoreType.DMA,
        ],
        compiler_params=_sc_compiler_params(),
    )
    return k(out_old, lg0, lg1, ea)


def _sc_compiler_params():
    cp = pltpu.CompilerParams()
    if "needs_layout_passes" in pltpu.CompilerParams.__dataclass_fields__:
        cp = dataclasses.replace(cp, needs_layout_passes=False)
    return cp


def _proj_body(x_ref, wu_ref, wv_ref, eu_ref, ev_ref):
    x = x_ref[...]
    eu_ref[...] = jax.lax.dot_general(
        x, wu_ref[...], (((1,), (1,)), ((), ())),
        preferred_element_type=jnp.float32)
    ev_ref[...] = jax.lax.dot_general(
        x, wv_ref[...], (((1,), (1,)), ((), ())),
        preferred_element_type=jnp.float32)


def _proj(x, Wu, Wv):
    blk = 2000
    grid = (N // blk,)
    return pl.pallas_call(
        _proj_body,
        grid=grid,
        in_specs=[
            pl.BlockSpec((blk, F), lambda i: (i, 0)),
            pl.BlockSpec((F, F), lambda i: (0, 0)),
            pl.BlockSpec((F, F), lambda i: (0, 0)),
        ],
        out_specs=[
            pl.BlockSpec((blk, F), lambda i: (i, 0)),
            pl.BlockSpec((blk, F), lambda i: (i, 0)),
        ],
        out_shape=[
            jax.ShapeDtypeStruct((N, F), jnp.float32),
            jax.ShapeDtypeStruct((N, F), jnp.float32),
        ],
    )(x, Wu, Wv)


def _batchnorm(x, g, b, eps=1e-5):
    m = jnp.mean(x, axis=0)
    v = jnp.var(x, axis=0)
    return (x - m) / jnp.sqrt(v + eps) * g + b


def _prelu(x, a):
    return jnp.where(x >= 0, x, a * x)


def _seg_softmax(scores, seg, num_segs):
    m = jax.ops.segment_max(scores, seg, num_segments=num_segs)
    m = jnp.where(jnp.isfinite(m), m, 0.0)
    e = jnp.exp(scores - m[seg])
    s = jax.ops.segment_sum(e, seg, num_segments=num_segs)
    return e / (s[seg] + 1e-16)


def kernel(x, edge_index, edge_attr, line_graph_edge_index, edge_index_batch, params):
    src, dst = edge_index[0], edge_index[1]
    lg = line_graph_edge_index
    batch = edge_index_batch
    eu, ev = _proj(x, params["Wu"], params["Wv"])
    euv = edge_attr @ params["We"].T
    ea = (eu[src] + ev[dst] + euv) / 3.0
    out = ea
    lg0, lg1 = lg[0], lg[1]

    def _step(out_c, _):
        out_n = _msg_pass(out_c, lg0, lg1, ea)
        sc = (out_n @ params["att_W"].T + params["att_b"])[:, 0]
        sc = _seg_softmax(sc, batch, G)
        gx = jax.ops.segment_sum(out_n * sc[:, None], batch, num_segments=G)
        gout = jnp.tanh(gx @ params["Wg"].T + params["bg"])
        return out_n, (out_n, gout)

    _, (outs3, gouts3) = jax.lax.scan(_step, out, None, length=T)
    gout_all = jnp.moveaxis(gouts3, 0, -1)
    out_all = jnp.moveaxis(outs3, 0, -1)
    scores = jnp.sum(gout_all * params["a"], axis=1, keepdims=True) + params["a_bias"]
    scores = jax.nn.softmax(scores, axis=-1)
    spe = scores[batch]
    o = jnp.sum(out_all * spe, axis=-1)
    x2 = x + jax.ops.segment_sum(o, dst, num_segments=N)
    p = params["blk"]
    out1 = _batchnorm(x2, p["bn0_g"], p["bn0_b"]) @ p["W1"].T + p["b1"]
    h = _prelu(_batchnorm(out1, p["bn2_g"], p["bn2_b"]), p["p3"]) @ p["W4"].T + p["b4"]
    out2 = (h + out1) / 2.0
    h = _prelu(_batchnorm(out2, p["bn5_g"], p["bn5_b"]), p["p6"]) @ p["W7"].T + p["b7"]
    out3 = (h + out2) / 2.0
    h = _prelu(_batchnorm(out3, p["bn8_g"], p["bn8_b"]), p["p9"]) @ p["W10"].T + p["b10"]
    out4 = (h + out3) / 2.0
    out5 = _prelu(_batchnorm(out4, p["bn11_g"], p["bn11_b"]), p["p12"]) @ p["W13"].T + p["b13"]
    return out5


# speedup vs baseline: 3.1637x; 3.1637x over previous
import dataclasses
import functools

import jax
import jax.numpy as jnp
from jax import lax
from jax.experimental import pallas as pl
from jax.experimental.pallas import tpu as pltpu
from jax.experimental.pallas import tpu_sc as plsc

N = 10000
F = 128
ED = 16
E = 320000
ELG = 640000
G = 256
T = 3
S = 6 * F

NC = 2
NS = 16
CH = 12800
NCHUNK = E // CH
NCK = (NCHUNK + NC - 1) // NC
PER_SUB = ELG // NS
IB = 2000
NBLK = PER_SUB // IB
NVEC = IB // 16
BB = 128
CAP = BB + 16
BINROWS = PER_SUB // BB + 2

BE = 6400
NBL = E // BE
NEG = -1e30


def _sc_compiler_params():
    cp = pltpu.CompilerParams()
    if "needs_layout_passes" in pltpu.CompilerParams.__dataclass_fields__:
        cp = dataclasses.replace(cp, needs_layout_passes=False)
    return cp


def _bin_kernel(out_hbm, lg0_hbm, lg1_hbm, ea_hbm,
                o_hbm, bs_hbm, bd_hbm, cnt_hbm,
                acc, l0v, l1v, sbuf, dbuf, fsrc, fdst, rows, cntv, gsem):
    cid = lax.axis_index("c")
    sid = lax.axis_index("s")
    z16 = jnp.zeros((16,), jnp.int32)
    ch16 = jnp.full((16,), CH, jnp.int32)

    def fill_scrap(lo):
        for k in range(lo, CAP // 16):
            sbuf[pl.ds(k * 16, 16)] = z16
            dbuf[pl.ds(k * 16, 16)] = ch16

    @pl.loop(0, NCK)
    def _chunk_loop(kc):
        chunk = cid * NCK + kc

        @pl.when(chunk < NCHUNK)
        def _(chunk=chunk):
            base = chunk * CH
            row0 = base + sid * (CH // NS)
            slot0 = (chunk * NS + sid) * BINROWS
            pltpu.sync_copy(ea_hbm.at[pl.ds(row0, CH // NS)],
                            acc.at[pl.ds(sid * (CH // NS), CH // NS)])
            plsc.subcore_barrier()
            fill_scrap(0)

            def flush(nf):
                for k in range(BB // 16):
                    sv = sbuf[pl.ds(k * 16, 16)]
                    dv = dbuf[pl.ds(k * 16, 16)]
                    fsrc[0, pl.ds(k * 16, 16)] = jnp.minimum(
                        jnp.maximum(sv, 0), E - 1)
                    fdst[0, pl.ds(k * 16, 16)] = jnp.minimum(
                        jnp.maximum(dv, 0), CH)
                pltpu.sync_copy(fsrc, bs_hbm.at[pl.ds(slot0 + nf, 1)])
                pltpu.sync_copy(fdst, bd_hbm.at[pl.ds(slot0 + nf, 1)])
                pltpu.async_copy(out_hbm.at[fsrc.at[0]], rows, gsem).wait()
                pltpu.sync_copy(rows, acc.at[fdst.at[0]], add=True)

            def scan_vec(v, carry, base=base):
                pos, nf = carry
                d = l1v[pl.ds(v * 16, 16)]
                s = l0v[pl.ds(v * 16, 16)]
                dl = d - base
                m = (dl >= 0) & (dl < CH)
                plsc.store_compressed(dbuf.at[pl.ds(pos, 16)], dl, mask=m)
                plsc.store_compressed(sbuf.at[pl.ds(pos, 16)], s, mask=m)
                pos = pos + jnp.sum(m.astype(jnp.int32))
                flushed = pos >= BB

                @pl.when(flushed)
                def _():
                    flush(nf)
                    rs = sbuf[pl.ds(BB, 16)]
                    rd = dbuf[pl.ds(BB, 16)]
                    sbuf[pl.ds(0, 16)] = rs
                    dbuf[pl.ds(0, 16)] = rd
                    fill_scrap(1)

                return (jnp.where(flushed, pos - BB, pos),
                        jnp.where(flushed, nf + 1, nf))

            def blk_body(b, carry):
                off = sid * PER_SUB + b * IB
                pltpu.sync_copy(lg0_hbm.at[pl.ds(off, IB)], l0v)
                pltpu.sync_copy(lg1_hbm.at[pl.ds(off, IB)], l1v)
                return lax.fori_loop(0, NVEC, scan_vec, carry)

            pos, nf = lax.fori_loop(0, NBLK, blk_body,
                                    (jnp.int32(0), jnp.int32(0)))
            flush(nf)
            cntv[pl.ds(0, 16)] = jnp.full((16,), 1, jnp.int32) * (nf + 1)
            pltpu.sync_copy(cntv,
                            cnt_hbm.at[pl.ds((chunk * NS + sid) * 16, 16)])
            plsc.subcore_barrier()
            pltpu.sync_copy(acc.at[pl.ds(sid * (CH // NS), CH // NS)],
                            o_hbm.at[pl.ds(row0, CH // NS)])
            plsc.subcore_barrier()


def _msg_pass_bin(out_old, lg0, lg1, ea):
    k = pl.kernel(
        _bin_kernel,
        out_type=[
            jax.ShapeDtypeStruct((E, F), jnp.float32),
            jax.ShapeDtypeStruct((NCHUNK * NS * BINROWS, BB), jnp.int32),
            jax.ShapeDtypeStruct((NCHUNK * NS * BINROWS, BB), jnp.int32),
            jax.ShapeDtypeStruct((NCHUNK * NS * 16,), jnp.int32),
        ],
        mesh=plsc.VectorSubcoreMesh(core_axis_name="c", subcore_axis_name="s"),
        scratch_types=[
            pltpu.VMEM_SHARED((CH + 8, F), jnp.float32),
            pltpu.VMEM((IB,), jnp.int32),
            pltpu.VMEM((IB,), jnp.int32),
            pltpu.VMEM((CAP,), jnp.int32),
            pltpu.VMEM((CAP,), jnp.int32),
            pltpu.VMEM((1, BB), jnp.int32),
            pltpu.VMEM((1, BB), jnp.int32),
            pltpu.VMEM((BB, F), jnp.float32),
            pltpu.VMEM((16,), jnp.int32),
            pltpu.SemaphoreType.DMA,
        ],
        compiler_params=_sc_compiler_params(),
    )
    return k(out_old, lg0, lg1, ea)


def _replay_kernel(out_hbm, ea_hbm, bs_hbm, bd_hbm, cnt_hbm,
                   o_hbm, acc, fsrc, fdst, rows, cntv, gsem):
    cid = lax.axis_index("c")
    sid = lax.axis_index("s")

    @pl.loop(0, NCK)
    def _chunk_loop(kc):
        chunk = cid * NCK + kc

        @pl.when(chunk < NCHUNK)
        def _(chunk=chunk):
            base = chunk * CH
            row0 = base + sid * (CH // NS)
            slot0 = (chunk * NS + sid) * BINROWS
            pltpu.sync_copy(ea_hbm.at[pl.ds(row0, CH // NS)],
                            acc.at[pl.ds(sid * (CH // NS), CH // NS)])
            pltpu.sync_copy(cnt_hbm.at[pl.ds((chunk * NS + sid) * 16, 16)],
                            cntv)
            plsc.subcore_barrier()
            nb = jnp.max(cntv[pl.ds(0, 16)])

            @pl.loop(0, nb)
            def _batch(j):
                pltpu.sync_copy(bs_hbm.at[pl.ds(slot0 + j, 1)], fsrc)
                pltpu.sync_copy(bd_hbm.at[pl.ds(slot0 + j, 1)], fdst)
                pltpu.async_copy(out_hbm.at[fsrc.at[0]], rows, gsem).wait()
                pltpu.sync_copy(rows, acc.at[fdst.at[0]], add=True)

            plsc.subcore_barrier()
            pltpu.sync_copy(acc.at[pl.ds(sid * (CH // NS), CH // NS)],
                            o_hbm.at[pl.ds(row0, CH // NS)])
            plsc.subcore_barrier()


def _msg_pass_replay(out_old, ea, bs, bd, cnt):
    k = pl.kernel(
        _replay_kernel,
        out_type=jax.ShapeDtypeStruct((E, F), jnp.float32),
        mesh=plsc.VectorSubcoreMesh(core_axis_name="c", subcore_axis_name="s"),
        scratch_types=[
            pltpu.VMEM_SHARED((CH + 8, F), jnp.float32),
            pltpu.VMEM((1, BB), jnp.int32),
            pltpu.VMEM((1, BB), jnp.int32),
            pltpu.VMEM((BB, F), jnp.float32),
            pltpu.VMEM((16,), jnp.int32),
            pltpu.SemaphoreType.DMA,
        ],
        compiler_params=_sc_compiler_params(),
    )
    return k(out_old, ea, bs, bd, cnt)


def _proj_body(x_ref, wu_ref, wv_ref, eu_ref, ev_ref):
    x = x_ref[...]
    eu_ref[...] = jax.lax.dot_general(
        x, wu_ref[...], (((1,), (1,)), ((), ())),
        preferred_element_type=jnp.float32)
    ev_ref[...] = jax.lax.dot_general(
        x, wv_ref[...], (((1,), (1,)), ((), ())),
        preferred_element_type=jnp.float32)


def _proj(x, Wu, Wv):
    blk = 2000
    return pl.pallas_call(
        _proj_body,
        grid=(N // blk,),
        in_specs=[
            pl.BlockSpec((blk, F), lambda i: (i, 0)),
            pl.BlockSpec((F, F), lambda i: (0, 0)),
            pl.BlockSpec((F, F), lambda i: (0, 0)),
        ],
        out_specs=[
            pl.BlockSpec((blk, F), lambda i: (i, 0)),
            pl.BlockSpec((blk, F), lambda i: (i, 0)),
        ],
        out_shape=[
            jax.ShapeDtypeStruct((N, F), jnp.float32),
            jax.ShapeDtypeStruct((N, F), jnp.float32),
        ],
    )(x, Wu, Wv)


def _attn_body(out_ref, b3_ref, attw_ref, attb_ref, gx_ref, m_ref, z_ref, a_ref):
    i = pl.program_id(0)

    @pl.when(i == 0)
    def _():
        m_ref[...] = jnp.full((G, 1), NEG, jnp.float32)
        z_ref[...] = jnp.zeros((G, 1), jnp.float32)
        a_ref[...] = jnp.zeros((G, F), jnp.float32)

    rows = out_ref[...]
    sT = jax.lax.dot_general(attw_ref[...], rows, (((1,), (1,)), ((), ())),
                             preferred_element_type=jnp.float32)
    sT = sT + attb_ref[0, 0]
    seg = b3_ref[0]
    ohT = seg == jax.lax.broadcasted_iota(jnp.int32, (G, BE), 0)
    ohfT = ohT.astype(jnp.float32)
    sbT = jnp.where(ohT, sT, NEG)
    bm = jnp.max(sbT, axis=1).reshape(G, 1)
    m_old = m_ref[...]
    m_new = jnp.maximum(m_old, bm)
    scale = jnp.exp(m_old - m_new)
    m_rowT = jax.lax.dot_general(m_new, ohfT, (((0,), (0,)), ((), ())),
                                 preferred_element_type=jnp.float32)
    e_rowT = jnp.exp(sT - m_rowT)
    ewT = ohfT * e_rowT
    z_ref[...] = z_ref[...] * scale + jnp.sum(ewT, axis=1).reshape(G, 1)
    a_ref[...] = a_ref[...] * scale + jax.lax.dot_general(
        ewT, rows, (((1,), (0,)), ((), ())),
        preferred_element_type=jnp.float32)
    m_ref[...] = m_new

    @pl.when(i == pl.num_programs(0) - 1)
    def _():
        gx_ref[...] = a_ref[...] / (z_ref[...] + 1e-16)


def _attn_pool(out, batch3, att_W, att_b):
    return pl.pallas_call(
        _attn_body,
        grid=(NBL,),
        in_specs=[
            pl.BlockSpec((BE, F), lambda i: (i, 0)),
            pl.BlockSpec((1, 1, BE), lambda i: (i, 0, 0)),
            pl.BlockSpec((1, F), lambda i: (0, 0)),
            pl.BlockSpec((1, 1), lambda i: (0, 0)),
        ],
        out_specs=pl.BlockSpec((G, F), lambda i: (0, 0)),
        out_shape=jax.ShapeDtypeStruct((G, F), jnp.float32),
        scratch_shapes=[
            pltpu.VMEM((G, 1), jnp.float32),
            pltpu.VMEM((G, 1), jnp.float32),
            pltpu.VMEM((G, F), jnp.float32),
        ],
    )(out, batch3, att_W, att_b)


def _batchnorm(x, g, b, eps=1e-5):
    m = jnp.mean(x, axis=0)
    v = jnp.var(x, axis=0)
    return (x - m) / jnp.sqrt(v + eps) * g + b


def _prelu(x, a):
    return jnp.where(x >= 0, x, a * x)


def kernel(x, edge_index, edge_attr, line_graph_edge_index, edge_index_batch, params):
    src, dst = edge_index[0], edge_index[1]
    lg0, lg1 = line_graph_edge_index[0], line_graph_edge_index[1]
    batch = edge_index_batch
    eu, ev = _proj(x, params["Wu"], params["Wv"])
    euv = edge_attr @ params["We"].T
    ea = (eu[src] + ev[dst] + euv) / 3.0

    batch3 = batch.reshape(NBL, 1, BE)
    attW = params["att_W"]
    attb = params["att_b"].reshape(1, 1)

    def _attn_stage(out_n):
        gx = _attn_pool(out_n, batch3, attW, attb)
        return jnp.tanh(gx @ params["Wg"].T + params["bg"])

    out1, bs, bd, cnt = _msg_pass_bin(ea, lg0, lg1, ea)
    gout1 = _attn_stage(out1)

    def _step(out_c, _):
        out_n = _msg_pass_replay(out_c, ea, bs, bd, cnt)
        return out_n, (out_n, _attn_stage(out_n))

    _, (outs23, gouts23) = jax.lax.scan(_step, out1, None, length=T - 1)
    out_all = jnp.moveaxis(jnp.concatenate([out1[None], outs23], 0), 0, -1)
    gout_all = jnp.moveaxis(jnp.concatenate([gout1[None], gouts23], 0), 0, -1)

    scores = jnp.sum(gout_all * params["a"], axis=1, keepdims=True) + params["a_bias"]
    scores = jax.nn.softmax(scores, axis=-1)
    spe = scores[batch]
    o = jnp.sum(out_all * spe, axis=-1)
    x2 = x + jax.ops.segment_sum(o, dst, num_segments=N)
    p = params["blk"]
    out1m = _batchnorm(x2, p["bn0_g"], p["bn0_b"]) @ p["W1"].T + p["b1"]
    h = _prelu(_batchnorm(out1m, p["bn2_g"], p["bn2_b"]), p["p3"]) @ p["W4"].T + p["b4"]
    out2 = (h + out1m) / 2.0
    h = _prelu(_batchnorm(out2, p["bn5_g"], p["bn5_b"]), p["p6"]) @ p["W7"].T + p["b7"]
    out3 = (h + out2) / 2.0
    h = _prelu(_batchnorm(out3, p["bn8_g"], p["bn8_b"]), p["p9"]) @ p["W10"].T + p["b10"]
    out4 = (h + out3) / 2.0
    out5 = _prelu(_batchnorm(out4, p["bn11_g"], p["bn11_b"]), p["p12"]) @ p["W13"].T + p["b13"]
    return out5


# speedup vs baseline: 3.2107x; 1.0149x over previous
import dataclasses
import functools

import jax
import jax.numpy as jnp
from jax import lax
from jax.experimental import pallas as pl
from jax.experimental.pallas import tpu as pltpu
from jax.experimental.pallas import tpu_sc as plsc

N = 10000
F = 128
ED = 16
E = 320000
ELG = 640000
G = 256
T = 3
S = 6 * F

NC = 2
NS = 16
CH = 12800
NCHUNK = E // CH
NCK = (NCHUNK + NC - 1) // NC
PER_SUB = ELG // NS
IB = 2000
NBLK = PER_SUB // IB
NVEC = IB // 16
BB = 128
CAP = BB + 16
BINROWS = PER_SUB // BB + 2

BE = 6400
NBL = E // BE
NEG = -1e30


def _sc_compiler_params():
    cp = pltpu.CompilerParams()
    if "needs_layout_passes" in pltpu.CompilerParams.__dataclass_fields__:
        cp = dataclasses.replace(cp, needs_layout_passes=False)
    return cp


def _bin_kernel(out_hbm, lg0_hbm, lg1_hbm, ea_hbm,
                o_hbm, bs_hbm, bd_hbm, cnt_hbm,
                acc, l0v, l1v, sbuf, dbuf, fsrc, fdst, rows, cntv, gsem):
    cid = lax.axis_index("c")
    sid = lax.axis_index("s")
    z16 = jnp.zeros((16,), jnp.int32)
    ch16 = jnp.full((16,), CH, jnp.int32)

    def fill_scrap(lo):
        for k in range(lo, CAP // 16):
            sbuf[pl.ds(k * 16, 16)] = z16
            dbuf[pl.ds(k * 16, 16)] = ch16

    @pl.loop(0, NCK)
    def _chunk_loop(kc):
        chunk = cid * NCK + kc

        @pl.when(chunk < NCHUNK)
        def _(chunk=chunk):
            base = chunk * CH
            row0 = base + sid * (CH // NS)
            slot0 = (chunk * NS + sid) * BINROWS
            pltpu.sync_copy(ea_hbm.at[pl.ds(row0, CH // NS)],
                            acc.at[pl.ds(sid * (CH // NS), CH // NS)])
            plsc.subcore_barrier()
            fill_scrap(0)

            def flush(nf):
                for k in range(BB // 16):
                    sv = sbuf[pl.ds(k * 16, 16)]
                    dv = dbuf[pl.ds(k * 16, 16)]
                    fsrc[0, pl.ds(k * 16, 16)] = jnp.minimum(
                        jnp.maximum(sv, 0), E - 1)
                    fdst[0, pl.ds(k * 16, 16)] = jnp.minimum(
                        jnp.maximum(dv, 0), CH)
                pltpu.sync_copy(fsrc, bs_hbm.at[pl.ds(slot0 + nf, 1)])
                pltpu.sync_copy(fdst, bd_hbm.at[pl.ds(slot0 + nf, 1)])
                pltpu.async_copy(out_hbm.at[fsrc.at[0]], rows, gsem).wait()
                pltpu.sync_copy(rows, acc.at[fdst.at[0]], add=True)

            def scan_vec(v, carry, base=base):
                pos, nf = carry
                d = l1v[pl.ds(v * 16, 16)]
                s = l0v[pl.ds(v * 16, 16)]
                dl = d - base
                m = (dl >= 0) & (dl < CH)
                plsc.store_compressed(dbuf.at[pl.ds(pos, 16)], dl, mask=m)
                plsc.store_compressed(sbuf.at[pl.ds(pos, 16)], s, mask=m)
                pos = pos + jnp.sum(m.astype(jnp.int32))
                flushed = pos >= BB

                @pl.when(flushed)
                def _():
                    flush(nf)
                    rs = sbuf[pl.ds(BB, 16)]
                    rd = dbuf[pl.ds(BB, 16)]
                    sbuf[pl.ds(0, 16)] = rs
                    dbuf[pl.ds(0, 16)] = rd
                    fill_scrap(1)

                return (jnp.where(flushed, pos - BB, pos),
                        jnp.where(flushed, nf + 1, nf))

            def blk_body(b, carry):
                off = sid * PER_SUB + b * IB
                pltpu.sync_copy(lg0_hbm.at[pl.ds(off, IB)], l0v)
                pltpu.sync_copy(lg1_hbm.at[pl.ds(off, IB)], l1v)
                return lax.fori_loop(0, NVEC, scan_vec, carry)

            pos, nf = lax.fori_loop(0, NBLK, blk_body,
                                    (jnp.int32(0), jnp.int32(0)))
            flush(nf)
            cntv[pl.ds(0, 16)] = jnp.full((16,), 1, jnp.int32) * (nf + 1)
            pltpu.sync_copy(cntv,
                            cnt_hbm.at[pl.ds((chunk * NS + sid) * 16, 16)])
            plsc.subcore_barrier()
            pltpu.sync_copy(acc.at[pl.ds(sid * (CH // NS), CH // NS)],
                            o_hbm.at[pl.ds(row0, CH // NS)])
            plsc.subcore_barrier()


def _msg_pass_bin(out_old, lg0, lg1, ea):
    k = pl.kernel(
        _bin_kernel,
        out_type=[
            jax.ShapeDtypeStruct((E, F), jnp.float32),
            jax.ShapeDtypeStruct((NCHUNK * NS * BINROWS, BB), jnp.int32),
            jax.ShapeDtypeStruct((NCHUNK * NS * BINROWS, BB), jnp.int32),
            jax.ShapeDtypeStruct((NCHUNK * NS * 16,), jnp.int32),
        ],
        mesh=plsc.VectorSubcoreMesh(core_axis_name="c", subcore_axis_name="s"),
        scratch_types=[
            pltpu.VMEM_SHARED((CH + 8, F), jnp.float32),
            pltpu.VMEM((IB,), jnp.int32),
            pltpu.VMEM((IB,), jnp.int32),
            pltpu.VMEM((CAP,), jnp.int32),
            pltpu.VMEM((CAP,), jnp.int32),
            pltpu.VMEM((1, BB), jnp.int32),
            pltpu.VMEM((1, BB), jnp.int32),
            pltpu.VMEM((BB, F), jnp.float32),
            pltpu.VMEM((16,), jnp.int32),
            pltpu.SemaphoreType.DMA,
        ],
        compiler_params=_sc_compiler_params(),
    )
    return k(out_old, lg0, lg1, ea)


def _replay_kernel(out_hbm, ea_hbm, bs_hbm, bd_hbm, cnt_hbm,
                   o_hbm, acc, fsrc, fdst, rows, cntv, gsem):
    cid = lax.axis_index("c")
    sid = lax.axis_index("s")

    @pl.loop(0, NCK)
    def _chunk_loop(kc):
        chunk = cid * NCK + kc

        @pl.when(chunk < NCHUNK)
        def _(chunk=chunk):
            base = chunk * CH
            row0 = base + sid * (CH // NS)
            slot0 = (chunk * NS + sid) * BINROWS
            pltpu.sync_copy(ea_hbm.at[pl.ds(row0, CH // NS)],
                            acc.at[pl.ds(sid * (CH // NS), CH // NS)])
            pltpu.sync_copy(cnt_hbm.at[pl.ds((chunk * NS + sid) * 16, 16)],
                            cntv)
            plsc.subcore_barrier()
            nb = jnp.max(cntv[pl.ds(0, 16)])

            @pl.loop(0, nb)
            def _batch(j):
                pltpu.sync_copy(bs_hbm.at[pl.ds(slot0 + j, 1)], fsrc)
                pltpu.sync_copy(bd_hbm.at[pl.ds(slot0 + j, 1)], fdst)
                pltpu.async_copy(out_hbm.at[fsrc.at[0]], rows, gsem).wait()
                pltpu.sync_copy(rows, acc.at[fdst.at[0]], add=True)

            plsc.subcore_barrier()
            pltpu.sync_copy(acc.at[pl.ds(sid * (CH // NS), CH // NS)],
                            o_hbm.at[pl.ds(row0, CH // NS)])
            plsc.subcore_barrier()


def _msg_pass_replay(out_old, ea, bs, bd, cnt):
    k = pl.kernel(
        _replay_kernel,
        out_type=jax.ShapeDtypeStruct((E, F), jnp.float32),
        mesh=plsc.VectorSubcoreMesh(core_axis_name="c", subcore_axis_name="s"),
        scratch_types=[
            pltpu.VMEM_SHARED((CH + 8, F), jnp.float32),
            pltpu.VMEM((1, BB), jnp.int32),
            pltpu.VMEM((1, BB), jnp.int32),
            pltpu.VMEM((BB, F), jnp.float32),
            pltpu.VMEM((16,), jnp.int32),
            pltpu.SemaphoreType.DMA,
        ],
        compiler_params=_sc_compiler_params(),
    )
    return k(out_old, ea, bs, bd, cnt)


IB2 = 80
EPS2 = E // (NC * NS)
NBLK2 = EPS2 // IB2
NROW2 = 624
NROW2L = N - 15 * NROW2


def _nscat_kernel(o_rows_hbm, dst_hbm, x_hbm, z_hbm, p_hbm,
                  acc, idxv, rowsv, gsem):
    cid = lax.axis_index("c")
    sid = lax.axis_index("s")

    def _share(copy):
        @pl.when(sid < 15)
        def _():
            copy(sid * NROW2, NROW2)

        @pl.when(sid == 15)
        def _():
            copy(15 * NROW2, NROW2L)

    @pl.when(cid == 0)
    def _():
        _share(lambda r0, n: pltpu.sync_copy(
            x_hbm.at[pl.ds(r0, n)], acc.at[pl.ds(r0, n)]))

    @pl.when(cid != 0)
    def _():
        _share(lambda r0, n: pltpu.sync_copy(
            z_hbm.at[pl.ds(0, n)], acc.at[pl.ds(r0, n)]))

    plsc.subcore_barrier()

    @pl.loop(0, NBLK2)
    def _blk(b):
        e0 = cid * (E // NC) + sid * EPS2 + b * IB2
        pltpu.sync_copy(dst_hbm.at[pl.ds(e0, IB2)], idxv)
        pltpu.async_copy(o_rows_hbm.at[pl.ds(e0, IB2)], rowsv, gsem).wait()
        pltpu.sync_copy(rowsv, acc.at[idxv], add=True)

    plsc.subcore_barrier()
    _share(lambda r0, n: pltpu.sync_copy(
        acc.at[pl.ds(r0, n)], p_hbm.at[cid, pl.ds(r0, n)]))


def _node_scatter(o_rows, dst, x, zeros):
    k = pl.kernel(
        _nscat_kernel,
        out_type=jax.ShapeDtypeStruct((NC, N, F), jnp.float32),
        mesh=plsc.VectorSubcoreMesh(core_axis_name="c", subcore_axis_name="s"),
        scratch_types=[
            pltpu.VMEM_SHARED((N + 8, F), jnp.float32),
            pltpu.VMEM((IB2,), jnp.int32),
            pltpu.VMEM((IB2, F), jnp.float32),
            pltpu.SemaphoreType.DMA,
        ],
        compiler_params=_sc_compiler_params(),
    )
    p = k(o_rows, dst, x, zeros)
    return p[0] + p[1]


def _proj_body(x_ref, wu_ref, wv_ref, eu_ref, ev_ref):
    x = x_ref[...]
    eu_ref[...] = jax.lax.dot_general(
        x, wu_ref[...], (((1,), (1,)), ((), ())),
        preferred_element_type=jnp.float32)
    ev_ref[...] = jax.lax.dot_general(
        x, wv_ref[...], (((1,), (1,)), ((), ())),
        preferred_element_type=jnp.float32)


def _proj(x, Wu, Wv):
    blk = 2000
    return pl.pallas_call(
        _proj_body,
        grid=(N // blk,),
        in_specs=[
            pl.BlockSpec((blk, F), lambda i: (i, 0)),
            pl.BlockSpec((F, F), lambda i: (0, 0)),
            pl.BlockSpec((F, F), lambda i: (0, 0)),
        ],
        out_specs=[
            pl.BlockSpec((blk, F), lambda i: (i, 0)),
            pl.BlockSpec((blk, F), lambda i: (i, 0)),
        ],
        out_shape=[
            jax.ShapeDtypeStruct((N, F), jnp.float32),
            jax.ShapeDtypeStruct((N, F), jnp.float32),
        ],
    )(x, Wu, Wv)


def _attn_body(out_ref, b3_ref, attw_ref, attb_ref, gx_ref, m_ref, z_ref, a_ref):
    i = pl.program_id(0)

    @pl.when(i == 0)
    def _():
        m_ref[...] = jnp.full((G, 1), NEG, jnp.float32)
        z_ref[...] = jnp.zeros((G, 1), jnp.float32)
        a_ref[...] = jnp.zeros((G, F), jnp.float32)

    rows = out_ref[...]
    sT = jax.lax.dot_general(attw_ref[...], rows, (((1,), (1,)), ((), ())),
                             preferred_element_type=jnp.float32)
    sT = sT + attb_ref[0, 0]
    seg = b3_ref[0]
    ohT = seg == jax.lax.broadcasted_iota(jnp.int32, (G, BE), 0)
    ohfT = ohT.astype(jnp.float32)
    sbT = jnp.where(ohT, sT, NEG)
    bm = jnp.max(sbT, axis=1).reshape(G, 1)
    m_old = m_ref[...]
    m_new = jnp.maximum(m_old, bm)
    scale = jnp.exp(m_old - m_new)
    m_rowT = jax.lax.dot_general(m_new, ohfT, (((0,), (0,)), ((), ())),
                                 preferred_element_type=jnp.float32)
    e_rowT = jnp.exp(sT - m_rowT)
    ewT = ohfT * e_rowT
    z_ref[...] = z_ref[...] * scale + jnp.sum(ewT, axis=1).reshape(G, 1)
    a_ref[...] = a_ref[...] * scale + jax.lax.dot_general(
        ewT, rows, (((1,), (0,)), ((), ())),
        preferred_element_type=jnp.float32)
    m_ref[...] = m_new

    @pl.when(i == pl.num_programs(0) - 1)
    def _():
        gx_ref[...] = a_ref[...] / (z_ref[...] + 1e-16)


def _attn_pool(out, batch3, att_W, att_b):
    return pl.pallas_call(
        _attn_body,
        grid=(NBL,),
        in_specs=[
            pl.BlockSpec((BE, F), lambda i: (i, 0)),
            pl.BlockSpec((1, 1, BE), lambda i: (i, 0, 0)),
            pl.BlockSpec((1, F), lambda i: (0, 0)),
            pl.BlockSpec((1, 1), lambda i: (0, 0)),
        ],
        out_specs=pl.BlockSpec((G, F), lambda i: (0, 0)),
        out_shape=jax.ShapeDtypeStruct((G, F), jnp.float32),
        scratch_shapes=[
            pltpu.VMEM((G, 1), jnp.float32),
            pltpu.VMEM((G, 1), jnp.float32),
            pltpu.VMEM((G, F), jnp.float32),
        ],
    )(out, batch3, att_W, att_b)


def _batchnorm(x, g, b, eps=1e-5):
    m = jnp.mean(x, axis=0)
    v = jnp.var(x, axis=0)
    return (x - m) / jnp.sqrt(v + eps) * g + b


def _prelu(x, a):
    return jnp.where(x >= 0, x, a * x)


def kernel(x, edge_index, edge_attr, line_graph_edge_index, edge_index_batch, params):
    src, dst = edge_index[0], edge_index[1]
    lg0, lg1 = line_graph_edge_index[0], line_graph_edge_index[1]
    batch = edge_index_batch
    eu, ev = _proj(x, params["Wu"], params["Wv"])
    euv = edge_attr @ params["We"].T
    ea = (eu[src] + ev[dst] + euv) / 3.0

    batch3 = batch.reshape(NBL, 1, BE)
    attW = params["att_W"]
    attb = params["att_b"].reshape(1, 1)

    def _attn_stage(out_n):
        gx = _attn_pool(out_n, batch3, attW, attb)
        return jnp.tanh(gx @ params["Wg"].T + params["bg"])

    out1, bs, bd, cnt = _msg_pass_bin(ea, lg0, lg1, ea)
    gout1 = _attn_stage(out1)

    def _step(out_c, _):
        out_n = _msg_pass_replay(out_c, ea, bs, bd, cnt)
        return out_n, (out_n, _attn_stage(out_n))

    _, (outs23, gouts23) = jax.lax.scan(_step, out1, None, length=T - 1)
    out_all = jnp.moveaxis(jnp.concatenate([out1[None], outs23], 0), 0, -1)
    gout_all = jnp.moveaxis(jnp.concatenate([gout1[None], gouts23], 0), 0, -1)

    scores = jnp.sum(gout_all * params["a"], axis=1, keepdims=True) + params["a_bias"]
    scores = jax.nn.softmax(scores, axis=-1)
    spe = scores[batch]
    o = jnp.sum(out_all * spe, axis=-1)
    x2 = _node_scatter(o, dst, x, jnp.zeros((NROW2L, F), jnp.float32))
    p = params["blk"]
    out1m = _batchnorm(x2, p["bn0_g"], p["bn0_b"]) @ p["W1"].T + p["b1"]
    h = _prelu(_batchnorm(out1m, p["bn2_g"], p["bn2_b"]), p["p3"]) @ p["W4"].T + p["b4"]
    out2 = (h + out1m) / 2.0
    h = _prelu(_batchnorm(out2, p["bn5_g"], p["bn5_b"]), p["p6"]) @ p["W7"].T + p["b7"]
    out3 = (h + out2) / 2.0
    h = _prelu(_batchnorm(out3, p["bn8_g"], p["bn8_b"]), p["p9"]) @ p["W10"].T + p["b10"]
    out4 = (h + out3) / 2.0
    out5 = _prelu(_batchnorm(out4, p["bn11_g"], p["bn11_b"]), p["p12"]) @ p["W13"].T + p["b13"]
    return out5


# speedup vs baseline: 3.4821x; 1.0845x over previous
import dataclasses
import functools

import jax
import jax.numpy as jnp
from jax import lax
from jax.experimental import pallas as pl
from jax.experimental.pallas import tpu as pltpu
from jax.experimental.pallas import tpu_sc as plsc

N = 10000
F = 128
ED = 16
E = 320000
ELG = 640000
G = 256
T = 3
S = 6 * F

NC = 2
NS = 16
CH = 12800
NCHUNK = E // CH
NCK = (NCHUNK + NC - 1) // NC
PER_SUB = ELG // NS
IB = 2000
NBLK = PER_SUB // IB
NVEC = IB // 16
BB = 128
CAP = BB + 16
BINROWS = PER_SUB // BB + 2

BE = 6400
NBL = E // BE
NEG = -1e30


def _sc_compiler_params():
    cp = pltpu.CompilerParams()
    if "needs_layout_passes" in pltpu.CompilerParams.__dataclass_fields__:
        cp = dataclasses.replace(cp, needs_layout_passes=False)
    return cp


def _bin_kernel(out_hbm, lg0_hbm, lg1_hbm, ea_hbm,
                o_hbm, bs_hbm, bd_hbm, cnt_hbm,
                acc, l0v, l1v, sbuf, dbuf, fsrc, fdst, rows, cntv, gsem):
    cid = lax.axis_index("c")
    sid = lax.axis_index("s")
    z16 = jnp.zeros((16,), jnp.int32)
    ch16 = jnp.full((16,), CH, jnp.int32)

    def fill_scrap(lo):
        for k in range(lo, CAP // 16):
            sbuf[pl.ds(k * 16, 16)] = z16
            dbuf[pl.ds(k * 16, 16)] = ch16

    @pl.loop(0, NCK)
    def _chunk_loop(kc):
        chunk = cid * NCK + kc

        @pl.when(chunk < NCHUNK)
        def _(chunk=chunk):
            base = chunk * CH
            row0 = base + sid * (CH // NS)
            slot0 = (chunk * NS + sid) * BINROWS
            pltpu.sync_copy(ea_hbm.at[pl.ds(row0, CH // NS)],
                            acc.at[pl.ds(sid * (CH // NS), CH // NS)])
            plsc.subcore_barrier()
            fill_scrap(0)

            def flush(nf):
                for k in range(BB // 16):
                    sv = sbuf[pl.ds(k * 16, 16)]
                    dv = dbuf[pl.ds(k * 16, 16)]
                    fsrc[0, pl.ds(k * 16, 16)] = jnp.minimum(
                        jnp.maximum(sv, 0), E - 1)
                    fdst[0, pl.ds(k * 16, 16)] = jnp.minimum(
                        jnp.maximum(dv, 0), CH)
                pltpu.sync_copy(fsrc, bs_hbm.at[pl.ds(slot0 + nf, 1)])
                pltpu.sync_copy(fdst, bd_hbm.at[pl.ds(slot0 + nf, 1)])
                pltpu.async_copy(out_hbm.at[fsrc.at[0]], rows, gsem).wait()
                pltpu.sync_copy(rows, acc.at[fdst.at[0]], add=True)

            def scan_vec(v, carry, base=base):
                pos, nf = carry
                d = l1v[pl.ds(v * 16, 16)]
                s = l0v[pl.ds(v * 16, 16)]
                dl = d - base
                m = (dl >= 0) & (dl < CH)
                plsc.store_compressed(dbuf.at[pl.ds(pos, 16)], dl, mask=m)
                plsc.store_compressed(sbuf.at[pl.ds(pos, 16)], s, mask=m)
                pos = pos + jnp.sum(m.astype(jnp.int32))
                flushed = pos >= BB

                @pl.when(flushed)
                def _():
                    flush(nf)
                    rs = sbuf[pl.ds(BB, 16)]
                    rd = dbuf[pl.ds(BB, 16)]
                    sbuf[pl.ds(0, 16)] = rs
                    dbuf[pl.ds(0, 16)] = rd
                    fill_scrap(1)

                return (jnp.where(flushed, pos - BB, pos),
                        jnp.where(flushed, nf + 1, nf))

            def blk_body(b, carry):
                off = sid * PER_SUB + b * IB
                pltpu.sync_copy(lg0_hbm.at[pl.ds(off, IB)], l0v)
                pltpu.sync_copy(lg1_hbm.at[pl.ds(off, IB)], l1v)
                return lax.fori_loop(0, NVEC, scan_vec, carry)

            pos, nf = lax.fori_loop(0, NBLK, blk_body,
                                    (jnp.int32(0), jnp.int32(0)))
            flush(nf)
            cntv[pl.ds(0, 16)] = jnp.full((16,), 1, jnp.int32) * (nf + 1)
            pltpu.sync_copy(cntv,
                            cnt_hbm.at[pl.ds((chunk * NS + sid) * 16, 16)])
            plsc.subcore_barrier()
            pltpu.sync_copy(acc.at[pl.ds(sid * (CH // NS), CH // NS)],
                            o_hbm.at[pl.ds(row0, CH // NS)])
            plsc.subcore_barrier()


def _msg_pass_bin(out_old, lg0, lg1, ea):
    k = pl.kernel(
        _bin_kernel,
        out_type=[
            jax.ShapeDtypeStruct((E, F), jnp.float32),
            jax.ShapeDtypeStruct((NCHUNK * NS * BINROWS, BB), jnp.int32),
            jax.ShapeDtypeStruct((NCHUNK * NS * BINROWS, BB), jnp.int32),
            jax.ShapeDtypeStruct((NCHUNK * NS * 16,), jnp.int32),
        ],
        mesh=plsc.VectorSubcoreMesh(core_axis_name="c", subcore_axis_name="s"),
        scratch_types=[
            pltpu.VMEM_SHARED((CH + 8, F), jnp.float32),
            pltpu.VMEM((IB,), jnp.int32),
            pltpu.VMEM((IB,), jnp.int32),
            pltpu.VMEM((CAP,), jnp.int32),
            pltpu.VMEM((CAP,), jnp.int32),
            pltpu.VMEM((1, BB), jnp.int32),
            pltpu.VMEM((1, BB), jnp.int32),
            pltpu.VMEM((BB, F), jnp.float32),
            pltpu.VMEM((16,), jnp.int32),
            pltpu.SemaphoreType.DMA,
        ],
        compiler_params=_sc_compiler_params(),
    )
    return k(out_old, lg0, lg1, ea)


def _replay_kernel(out_hbm, ea_hbm, bs_hbm, bd_hbm, cnt_hbm,
                   o_hbm, acc, fsrc, fdst, rows, cntv, gsem):
    cid = lax.axis_index("c")
    sid = lax.axis_index("s")

    @pl.loop(0, NCK)
    def _chunk_loop(kc):
        chunk = cid * NCK + kc

        @pl.when(chunk < NCHUNK)
        def _(chunk=chunk):
            base = chunk * CH
            row0 = base + sid * (CH // NS)
            slot0 = (chunk * NS + sid) * BINROWS
            pltpu.sync_copy(ea_hbm.at[pl.ds(row0, CH // NS)],
                            acc.at[pl.ds(sid * (CH // NS), CH // NS)])
            pltpu.sync_copy(cnt_hbm.at[pl.ds((chunk * NS + sid) * 16, 16)],
                            cntv)
            plsc.subcore_barrier()
            nb = jnp.max(cntv[pl.ds(0, 16)])

            @pl.loop(0, nb)
            def _batch(j):
                pltpu.sync_copy(bs_hbm.at[pl.ds(slot0 + j, 1)], fsrc)
                pltpu.sync_copy(bd_hbm.at[pl.ds(slot0 + j, 1)], fdst)
                pltpu.async_copy(out_hbm.at[fsrc.at[0]], rows, gsem).wait()
                pltpu.sync_copy(rows, acc.at[fdst.at[0]], add=True)

            plsc.subcore_barrier()
            pltpu.sync_copy(acc.at[pl.ds(sid * (CH // NS), CH // NS)],
                            o_hbm.at[pl.ds(row0, CH // NS)])
            plsc.subcore_barrier()


def _msg_pass_replay(out_old, ea, bs, bd, cnt):
    k = pl.kernel(
        _replay_kernel,
        out_type=jax.ShapeDtypeStruct((E, F), jnp.float32),
        mesh=plsc.VectorSubcoreMesh(core_axis_name="c", subcore_axis_name="s"),
        scratch_types=[
            pltpu.VMEM_SHARED((CH + 8, F), jnp.float32),
            pltpu.VMEM((1, BB), jnp.int32),
            pltpu.VMEM((1, BB), jnp.int32),
            pltpu.VMEM((BB, F), jnp.float32),
            pltpu.VMEM((16,), jnp.int32),
            pltpu.SemaphoreType.DMA,
        ],
        compiler_params=_sc_compiler_params(),
    )
    return k(out_old, ea, bs, bd, cnt)


IB2 = 80
EPS2 = E // (NC * NS)
NBLK2 = EPS2 // IB2
NROW2 = 624
NROW2L = N - 15 * NROW2


def _nscat_kernel(o_rows_hbm, dst_hbm, x_hbm, z_hbm, p_hbm,
                  acc, idxv, rowsv, gsem):
    cid = lax.axis_index("c")
    sid = lax.axis_index("s")

    def _share(copy):
        @pl.when(sid < 15)
        def _():
            copy(sid * NROW2, NROW2)

        @pl.when(sid == 15)
        def _():
            copy(15 * NROW2, NROW2L)

    @pl.when(cid == 0)
    def _():
        _share(lambda r0, n: pltpu.sync_copy(
            x_hbm.at[pl.ds(r0, n)], acc.at[pl.ds(r0, n)]))

    @pl.when(cid != 0)
    def _():
        _share(lambda r0, n: pltpu.sync_copy(
            z_hbm.at[pl.ds(0, n)], acc.at[pl.ds(r0, n)]))

    plsc.subcore_barrier()

    @pl.loop(0, NBLK2)
    def _blk(b):
        e0 = cid * (E // NC) + sid * EPS2 + b * IB2
        pltpu.sync_copy(dst_hbm.at[pl.ds(e0, IB2)], idxv)
        pltpu.async_copy(o_rows_hbm.at[pl.ds(e0, IB2)], rowsv, gsem).wait()
        pltpu.sync_copy(rowsv, acc.at[idxv], add=True)

    plsc.subcore_barrier()
    _share(lambda r0, n: pltpu.sync_copy(
        acc.at[pl.ds(r0, n)], p_hbm.at[cid, pl.ds(r0, n)]))


def _node_scatter(o_rows, dst, x, zeros):
    k = pl.kernel(
        _nscat_kernel,
        out_type=jax.ShapeDtypeStruct((NC, N, F), jnp.float32),
        mesh=plsc.VectorSubcoreMesh(core_axis_name="c", subcore_axis_name="s"),
        scratch_types=[
            pltpu.VMEM_SHARED((N + 8, F), jnp.float32),
            pltpu.VMEM((IB2,), jnp.int32),
            pltpu.VMEM((IB2, F), jnp.float32),
            pltpu.SemaphoreType.DMA,
        ],
        compiler_params=_sc_compiler_params(),
    )
    p = k(o_rows, dst, x, zeros)
    return p[0] + p[1]


def _eab_kernel(eu_hbm, ev_hbm, src_hbm, dst_hbm, uv_hbm, ea_hbm,
                sidx, didx, eur, evr, uvr, sem1, sem2):
    cid = lax.axis_index("c")
    sid = lax.axis_index("s")
    third = jnp.float32(1.0 / 3.0)

    @pl.loop(0, NBLK2)
    def _blk(b):
        e0 = cid * (E // NC) + sid * EPS2 + b * IB2
        pltpu.sync_copy(src_hbm.at[pl.ds(e0, IB2)], sidx)
        pltpu.sync_copy(dst_hbm.at[pl.ds(e0, IB2)], didx)
        c1 = pltpu.async_copy(eu_hbm.at[sidx], eur, sem1)
        c2 = pltpu.async_copy(ev_hbm.at[didx], evr, sem2)
        pltpu.sync_copy(uv_hbm.at[pl.ds(e0, IB2)], uvr)
        c1.wait()
        c2.wait()

        @pl.loop(0, IB2)
        def _row(r):
            for k in range(F // 16):
                sl = pl.ds(k * 16, 16)
                uvr[r, sl] = (eur[r, sl] + evr[r, sl] + uvr[r, sl]) * third

        pltpu.sync_copy(uvr, ea_hbm.at[pl.ds(e0, IB2)])


def _ea_build(eu, ev, src, dst, euv):
    k = pl.kernel(
        _eab_kernel,
        out_type=jax.ShapeDtypeStruct((E, F), jnp.float32),
        mesh=plsc.VectorSubcoreMesh(core_axis_name="c", subcore_axis_name="s"),
        scratch_types=[
            pltpu.VMEM((IB2,), jnp.int32),
            pltpu.VMEM((IB2,), jnp.int32),
            pltpu.VMEM((IB2, F), jnp.float32),
            pltpu.VMEM((IB2, F), jnp.float32),
            pltpu.VMEM((IB2, F), jnp.float32),
            pltpu.SemaphoreType.DMA,
            pltpu.SemaphoreType.DMA,
        ],
        compiler_params=_sc_compiler_params(),
    )
    return k(eu, ev, src, dst, euv)


def _proj_body(x_ref, wu_ref, wv_ref, eu_ref, ev_ref):
    x = x_ref[...]
    eu_ref[...] = jax.lax.dot_general(
        x, wu_ref[...], (((1,), (1,)), ((), ())),
        preferred_element_type=jnp.float32)
    ev_ref[...] = jax.lax.dot_general(
        x, wv_ref[...], (((1,), (1,)), ((), ())),
        preferred_element_type=jnp.float32)


def _proj(x, Wu, Wv):
    blk = 2000
    return pl.pallas_call(
        _proj_body,
        grid=(N // blk,),
        in_specs=[
            pl.BlockSpec((blk, F), lambda i: (i, 0)),
            pl.BlockSpec((F, F), lambda i: (0, 0)),
            pl.BlockSpec((F, F), lambda i: (0, 0)),
        ],
        out_specs=[
            pl.BlockSpec((blk, F), lambda i: (i, 0)),
            pl.BlockSpec((blk, F), lambda i: (i, 0)),
        ],
        out_shape=[
            jax.ShapeDtypeStruct((N, F), jnp.float32),
            jax.ShapeDtypeStruct((N, F), jnp.float32),
        ],
    )(x, Wu, Wv)


def _attn_body(out_ref, b3_ref, attw_ref, attb_ref, gx_ref, m_ref, z_ref, a_ref):
    i = pl.program_id(0)

    @pl.when(i == 0)
    def _():
        m_ref[...] = jnp.full((G, 1), NEG, jnp.float32)
        z_ref[...] = jnp.zeros((G, 1), jnp.float32)
        a_ref[...] = jnp.zeros((G, F), jnp.float32)

    rows = out_ref[...]
    sT = jax.lax.dot_general(attw_ref[...], rows, (((1,), (1,)), ((), ())),
                             preferred_element_type=jnp.float32)
    sT = sT + attb_ref[0, 0]
    seg = b3_ref[0]
    ohT = seg == jax.lax.broadcasted_iota(jnp.int32, (G, BE), 0)
    ohfT = ohT.astype(jnp.float32)
    sbT = jnp.where(ohT, sT, NEG)
    bm = jnp.max(sbT, axis=1).reshape(G, 1)
    m_old = m_ref[...]
    m_new = jnp.maximum(m_old, bm)
    scale = jnp.exp(m_old - m_new)
    m_rowT = jax.lax.dot_general(m_new, ohfT, (((0,), (0,)), ((), ())),
                                 preferred_element_type=jnp.float32)
    e_rowT = jnp.exp(sT - m_rowT)
    ewT = ohfT * e_rowT
    z_ref[...] = z_ref[...] * scale + jnp.sum(ewT, axis=1).reshape(G, 1)
    a_ref[...] = a_ref[...] * scale + jax.lax.dot_general(
        ewT, rows, (((1,), (0,)), ((), ())),
        preferred_element_type=jnp.float32)
    m_ref[...] = m_new

    @pl.when(i == pl.num_programs(0) - 1)
    def _():
        gx_ref[...] = a_ref[...] / (z_ref[...] + 1e-16)


def _attn_pool(out, batch3, att_W, att_b):
    return pl.pallas_call(
        _attn_body,
        grid=(NBL,),
        in_specs=[
            pl.BlockSpec((BE, F), lambda i: (i, 0)),
            pl.BlockSpec((1, 1, BE), lambda i: (i, 0, 0)),
            pl.BlockSpec((1, F), lambda i: (0, 0)),
            pl.BlockSpec((1, 1), lambda i: (0, 0)),
        ],
        out_specs=pl.BlockSpec((G, F), lambda i: (0, 0)),
        out_shape=jax.ShapeDtypeStruct((G, F), jnp.float32),
        scratch_shapes=[
            pltpu.VMEM((G, 1), jnp.float32),
            pltpu.VMEM((G, 1), jnp.float32),
            pltpu.VMEM((G, F), jnp.float32),
        ],
    )(out, batch3, att_W, att_b)


def _batchnorm(x, g, b, eps=1e-5):
    m = jnp.mean(x, axis=0)
    v = jnp.var(x, axis=0)
    return (x - m) / jnp.sqrt(v + eps) * g + b


def _prelu(x, a):
    return jnp.where(x >= 0, x, a * x)


def kernel(x, edge_index, edge_attr, line_graph_edge_index, edge_index_batch, params):
    src, dst = edge_index[0], edge_index[1]
    lg0, lg1 = line_graph_edge_index[0], line_graph_edge_index[1]
    batch = edge_index_batch
    eu, ev = _proj(x, params["Wu"], params["Wv"])
    euv = edge_attr @ params["We"].T
    ea = _ea_build(eu, ev, src, dst, euv)

    batch3 = batch.reshape(NBL, 1, BE)
    attW = params["att_W"]
    attb = params["att_b"].reshape(1, 1)

    def _attn_stage(out_n):
        gx = _attn_pool(out_n, batch3, attW, attb)
        return jnp.tanh(gx @ params["Wg"].T + params["bg"])

    out1, bs, bd, cnt = _msg_pass_bin(ea, lg0, lg1, ea)
    gout1 = _attn_stage(out1)

    def _step(out_c, _):
        out_n = _msg_pass_replay(out_c, ea, bs, bd, cnt)
        return out_n, (out_n, _attn_stage(out_n))

    _, (outs23, gouts23) = jax.lax.scan(_step, out1, None, length=T - 1)
    out_all = jnp.moveaxis(jnp.concatenate([out1[None], outs23], 0), 0, -1)
    gout_all = jnp.moveaxis(jnp.concatenate([gout1[None], gouts23], 0), 0, -1)

    scores = jnp.sum(gout_all * params["a"], axis=1, keepdims=True) + params["a_bias"]
    scores = jax.nn.softmax(scores, axis=-1)
    spe = scores[batch]
    o = jnp.sum(out_all * spe, axis=-1)
    x2 = _node_scatter(o, dst, x, jnp.zeros((NROW2L, F), jnp.float32))
    p = params["blk"]
    out1m = _batchnorm(x2, p["bn0_g"], p["bn0_b"]) @ p["W1"].T + p["b1"]
    h = _prelu(_batchnorm(out1m, p["bn2_g"], p["bn2_b"]), p["p3"]) @ p["W4"].T + p["b4"]
    out2 = (h + out1m) / 2.0
    h = _prelu(_batchnorm(out2, p["bn5_g"], p["bn5_b"]), p["p6"]) @ p["W7"].T + p["b7"]
    out3 = (h + out2) / 2.0
    h = _prelu(_batchnorm(out3, p["bn8_g"], p["bn8_b"]), p["p9"]) @ p["W10"].T + p["b10"]
    out4 = (h + out3) / 2.0
    out5 = _prelu(_batchnorm(out4, p["bn11_g"], p["bn11_b"]), p["p12"]) @ p["W13"].T + p["b13"]
    return out5


# speedup vs baseline: 3.8167x; 1.0961x over previous
import dataclasses
import functools

import jax
import jax.numpy as jnp
from jax import lax
from jax.experimental import pallas as pl
from jax.experimental.pallas import tpu as pltpu
from jax.experimental.pallas import tpu_sc as plsc

N = 10000
F = 128
ED = 16
E = 320000
ELG = 640000
G = 256
T = 3
S = 6 * F

NC = 2
NS = 16
CH = 12800
NCHUNK = E // CH
NCK = (NCHUNK + NC - 1) // NC
PER_SUB = ELG // NS
IB = 2000
NBLK = PER_SUB // IB
NVEC = IB // 16
BB = 128
CAP = BB + 16
BINROWS = PER_SUB // BB + 2

BE = 6400
NBL = E // BE
NEG = -1e30


def _sc_compiler_params():
    cp = pltpu.CompilerParams()
    if "needs_layout_passes" in pltpu.CompilerParams.__dataclass_fields__:
        cp = dataclasses.replace(cp, needs_layout_passes=False)
    return cp


def _bin_kernel(out_hbm, lg0_hbm, lg1_hbm, ea_hbm,
                o_hbm, bs_hbm, bd_hbm, cnt_hbm,
                acc, l0v, l1v, sbuf, dbuf, fsrc, fdst, rows, cntv, gsem):
    cid = lax.axis_index("c")
    sid = lax.axis_index("s")
    z16 = jnp.zeros((16,), jnp.int32)
    ch16 = jnp.full((16,), CH, jnp.int32)

    def fill_scrap(lo):
        for k in range(lo, CAP // 16):
            sbuf[pl.ds(k * 16, 16)] = z16
            dbuf[pl.ds(k * 16, 16)] = ch16

    @pl.loop(0, NCK)
    def _chunk_loop(kc):
        chunk = cid * NCK + kc

        @pl.when(chunk < NCHUNK)
        def _(chunk=chunk):
            base = chunk * CH
            row0 = base + sid * (CH // NS)
            slot0 = (chunk * NS + sid) * BINROWS
            pltpu.sync_copy(ea_hbm.at[pl.ds(row0, CH // NS)],
                            acc.at[pl.ds(sid * (CH // NS), CH // NS)])
            plsc.subcore_barrier()
            fill_scrap(0)

            def flush(nf):
                for k in range(BB // 16):
                    sv = sbuf[pl.ds(k * 16, 16)]
                    dv = dbuf[pl.ds(k * 16, 16)]
                    fsrc[0, pl.ds(k * 16, 16)] = jnp.minimum(
                        jnp.maximum(sv, 0), E - 1)
                    fdst[0, pl.ds(k * 16, 16)] = jnp.minimum(
                        jnp.maximum(dv, 0), CH)
                pltpu.sync_copy(fsrc, bs_hbm.at[pl.ds(slot0 + nf, 1)])
                pltpu.sync_copy(fdst, bd_hbm.at[pl.ds(slot0 + nf, 1)])
                pltpu.async_copy(out_hbm.at[fsrc.at[0]], rows, gsem).wait()
                pltpu.sync_copy(rows, acc.at[fdst.at[0]], add=True)

            def scan_vec(v, carry, base=base):
                pos, nf = carry
                d = l1v[pl.ds(v * 16, 16)]
                s = l0v[pl.ds(v * 16, 16)]
                dl = d - base
                m = (dl >= 0) & (dl < CH)
                plsc.store_compressed(dbuf.at[pl.ds(pos, 16)], dl, mask=m)
                plsc.store_compressed(sbuf.at[pl.ds(pos, 16)], s, mask=m)
                pos = pos + jnp.sum(m.astype(jnp.int32))
                flushed = pos >= BB

                @pl.when(flushed)
                def _():
                    flush(nf)
                    rs = sbuf[pl.ds(BB, 16)]
                    rd = dbuf[pl.ds(BB, 16)]
                    sbuf[pl.ds(0, 16)] = rs
                    dbuf[pl.ds(0, 16)] = rd
                    fill_scrap(1)

                return (jnp.where(flushed, pos - BB, pos),
                        jnp.where(flushed, nf + 1, nf))

            def blk_body(b, carry):
                off = sid * PER_SUB + b * IB
                pltpu.sync_copy(lg0_hbm.at[pl.ds(off, IB)], l0v)
                pltpu.sync_copy(lg1_hbm.at[pl.ds(off, IB)], l1v)
                return lax.fori_loop(0, NVEC, scan_vec, carry)

            pos, nf = lax.fori_loop(0, NBLK, blk_body,
                                    (jnp.int32(0), jnp.int32(0)))
            flush(nf)
            cntv[pl.ds(0, 16)] = jnp.full((16,), 1, jnp.int32) * (nf + 1)
            pltpu.sync_copy(cntv,
                            cnt_hbm.at[pl.ds((chunk * NS + sid) * 16, 16)])
            plsc.subcore_barrier()
            pltpu.sync_copy(acc.at[pl.ds(sid * (CH // NS), CH // NS)],
                            o_hbm.at[pl.ds(row0, CH // NS)])
            plsc.subcore_barrier()


def _msg_pass_bin(out_old, lg0, lg1, ea):
    k = pl.kernel(
        _bin_kernel,
        out_type=[
            jax.ShapeDtypeStruct((E, F), jnp.float32),
            jax.ShapeDtypeStruct((NCHUNK * NS * BINROWS, BB), jnp.int32),
            jax.ShapeDtypeStruct((NCHUNK * NS * BINROWS, BB), jnp.int32),
            jax.ShapeDtypeStruct((NCHUNK * NS * 16,), jnp.int32),
        ],
        mesh=plsc.VectorSubcoreMesh(core_axis_name="c", subcore_axis_name="s"),
        scratch_types=[
            pltpu.VMEM_SHARED((CH + 8, F), jnp.float32),
            pltpu.VMEM((IB,), jnp.int32),
            pltpu.VMEM((IB,), jnp.int32),
            pltpu.VMEM((CAP,), jnp.int32),
            pltpu.VMEM((CAP,), jnp.int32),
            pltpu.VMEM((1, BB), jnp.int32),
            pltpu.VMEM((1, BB), jnp.int32),
            pltpu.VMEM((BB, F), jnp.float32),
            pltpu.VMEM((16,), jnp.int32),
            pltpu.SemaphoreType.DMA,
        ],
        compiler_params=_sc_compiler_params(),
    )
    return k(out_old, lg0, lg1, ea)


def _replay_kernel(out_hbm, ea_hbm, bs_hbm, bd_hbm, cnt_hbm,
                   o_hbm, acc, fsrc, fdst, rows, cntv, gsem):
    cid = lax.axis_index("c")
    sid = lax.axis_index("s")

    @pl.loop(0, NCK)
    def _chunk_loop(kc):
        chunk = cid * NCK + kc

        @pl.when(chunk < NCHUNK)
        def _(chunk=chunk):
            base = chunk * CH
            row0 = base + sid * (CH // NS)
            slot0 = (chunk * NS + sid) * BINROWS
            pltpu.sync_copy(ea_hbm.at[pl.ds(row0, CH // NS)],
                            acc.at[pl.ds(sid * (CH // NS), CH // NS)])
            pltpu.sync_copy(cnt_hbm.at[pl.ds((chunk * NS + sid) * 16, 16)],
                            cntv)
            plsc.subcore_barrier()
            nb = jnp.max(cntv[pl.ds(0, 16)])

            @pl.loop(0, nb)
            def _batch(j):
                pltpu.sync_copy(bs_hbm.at[pl.ds(slot0 + j, 1)], fsrc)
                pltpu.sync_copy(bd_hbm.at[pl.ds(slot0 + j, 1)], fdst)
                pltpu.async_copy(out_hbm.at[fsrc.at[0]], rows, gsem).wait()
                pltpu.sync_copy(rows, acc.at[fdst.at[0]], add=True)

            plsc.subcore_barrier()
            pltpu.sync_copy(acc.at[pl.ds(sid * (CH // NS), CH // NS)],
                            o_hbm.at[pl.ds(row0, CH // NS)])
            plsc.subcore_barrier()


def _msg_pass_replay(out_old, ea, bs, bd, cnt):
    k = pl.kernel(
        _replay_kernel,
        out_type=jax.ShapeDtypeStruct((E, F), jnp.float32),
        mesh=plsc.VectorSubcoreMesh(core_axis_name="c", subcore_axis_name="s"),
        scratch_types=[
            pltpu.VMEM_SHARED((CH + 8, F), jnp.float32),
            pltpu.VMEM((1, BB), jnp.int32),
            pltpu.VMEM((1, BB), jnp.int32),
            pltpu.VMEM((BB, F), jnp.float32),
            pltpu.VMEM((16,), jnp.int32),
            pltpu.SemaphoreType.DMA,
        ],
        compiler_params=_sc_compiler_params(),
    )
    return k(out_old, ea, bs, bd, cnt)


IB2 = 80
EPS2 = E // (NC * NS)
NBLK2 = EPS2 // IB2
NROW2 = 624
NROW2L = N - 15 * NROW2


def _nscat_kernel(o_rows_hbm, dst_hbm, x_hbm, z_hbm, p_hbm,
                  acc, idxv, rowsv, gsem):
    cid = lax.axis_index("c")
    sid = lax.axis_index("s")

    def _share(copy):
        @pl.when(sid < 15)
        def _():
            copy(sid * NROW2, NROW2)

        @pl.when(sid == 15)
        def _():
            copy(15 * NROW2, NROW2L)

    @pl.when(cid == 0)
    def _():
        _share(lambda r0, n: pltpu.sync_copy(
            x_hbm.at[pl.ds(r0, n)], acc.at[pl.ds(r0, n)]))

    @pl.when(cid != 0)
    def _():
        _share(lambda r0, n: pltpu.sync_copy(
            z_hbm.at[pl.ds(0, n)], acc.at[pl.ds(r0, n)]))

    plsc.subcore_barrier()

    @pl.loop(0, NBLK2)
    def _blk(b):
        e0 = cid * (E // NC) + sid * EPS2 + b * IB2
        pltpu.sync_copy(dst_hbm.at[pl.ds(e0, IB2)], idxv)
        pltpu.async_copy(o_rows_hbm.at[pl.ds(e0, IB2)], rowsv, gsem).wait()
        pltpu.sync_copy(rowsv, acc.at[idxv], add=True)

    plsc.subcore_barrier()
    _share(lambda r0, n: pltpu.sync_copy(
        acc.at[pl.ds(r0, n)], p_hbm.at[cid, pl.ds(r0, n)]))


def _node_scatter(o_rows, dst, x, zeros):
    k = pl.kernel(
        _nscat_kernel,
        out_type=jax.ShapeDtypeStruct((NC, N, F), jnp.float32),
        mesh=plsc.VectorSubcoreMesh(core_axis_name="c", subcore_axis_name="s"),
        scratch_types=[
            pltpu.VMEM_SHARED((N + 8, F), jnp.float32),
            pltpu.VMEM((IB2,), jnp.int32),
            pltpu.VMEM((IB2, F), jnp.float32),
            pltpu.SemaphoreType.DMA,
        ],
        compiler_params=_sc_compiler_params(),
    )
    p = k(o_rows, dst, x, zeros)
    return p[0] + p[1]


def _eab_kernel(eu_hbm, ev_hbm, src_hbm, dst_hbm, uv_hbm, ea_hbm,
                sidx, didx, eur, evr, uvr, sem1, sem2):
    cid = lax.axis_index("c")
    sid = lax.axis_index("s")
    third = jnp.float32(1.0 / 3.0)

    @pl.loop(0, NBLK2)
    def _blk(b):
        e0 = cid * (E // NC) + sid * EPS2 + b * IB2
        pltpu.sync_copy(src_hbm.at[pl.ds(e0, IB2)], sidx)
        pltpu.sync_copy(dst_hbm.at[pl.ds(e0, IB2)], didx)
        c1 = pltpu.async_copy(eu_hbm.at[sidx], eur, sem1)
        c2 = pltpu.async_copy(ev_hbm.at[didx], evr, sem2)
        pltpu.sync_copy(uv_hbm.at[pl.ds(e0, IB2)], uvr)
        c1.wait()
        c2.wait()

        @pl.loop(0, IB2)
        def _row(r):
            for k in range(F // 16):
                sl = pl.ds(k * 16, 16)
                uvr[r, sl] = (eur[r, sl] + evr[r, sl] + uvr[r, sl]) * third

        pltpu.sync_copy(uvr, ea_hbm.at[pl.ds(e0, IB2)])


def _ea_build(eu, ev, src, dst, euv):
    k = pl.kernel(
        _eab_kernel,
        out_type=jax.ShapeDtypeStruct((E, F), jnp.float32),
        mesh=plsc.VectorSubcoreMesh(core_axis_name="c", subcore_axis_name="s"),
        scratch_types=[
            pltpu.VMEM((IB2,), jnp.int32),
            pltpu.VMEM((IB2,), jnp.int32),
            pltpu.VMEM((IB2, F), jnp.float32),
            pltpu.VMEM((IB2, F), jnp.float32),
            pltpu.VMEM((IB2, F), jnp.float32),
            pltpu.SemaphoreType.DMA,
            pltpu.SemaphoreType.DMA,
        ],
        compiler_params=_sc_compiler_params(),
    )
    return k(eu, ev, src, dst, euv)


def _proj_body(x_ref, wu_ref, wv_ref, eu_ref, ev_ref):
    x = x_ref[...]
    eu_ref[...] = jax.lax.dot_general(
        x, wu_ref[...], (((1,), (1,)), ((), ())),
        preferred_element_type=jnp.float32)
    ev_ref[...] = jax.lax.dot_general(
        x, wv_ref[...], (((1,), (1,)), ((), ())),
        preferred_element_type=jnp.float32)


def _proj(x, Wu, Wv):
    blk = 2000
    return pl.pallas_call(
        _proj_body,
        grid=(N // blk,),
        in_specs=[
            pl.BlockSpec((blk, F), lambda i: (i, 0)),
            pl.BlockSpec((F, F), lambda i: (0, 0)),
            pl.BlockSpec((F, F), lambda i: (0, 0)),
        ],
        out_specs=[
            pl.BlockSpec((blk, F), lambda i: (i, 0)),
            pl.BlockSpec((blk, F), lambda i: (i, 0)),
        ],
        out_shape=[
            jax.ShapeDtypeStruct((N, F), jnp.float32),
            jax.ShapeDtypeStruct((N, F), jnp.float32),
        ],
    )(x, Wu, Wv)


def _attn_body(out_ref, b3_ref, attw_ref, attb_ref, gx_ref, m_ref, z_ref, a_ref):
    i = pl.program_id(0)

    @pl.when(i == 0)
    def _():
        m_ref[...] = jnp.full((G, 1), NEG, jnp.float32)
        z_ref[...] = jnp.zeros((G, 1), jnp.float32)
        a_ref[...] = jnp.zeros((G, F), jnp.float32)

    rows = out_ref[...]
    sT = jax.lax.dot_general(attw_ref[...], rows, (((1,), (1,)), ((), ())),
                             preferred_element_type=jnp.float32)
    sT = sT + attb_ref[0, 0]
    seg = b3_ref[0]
    ohT = seg == jax.lax.broadcasted_iota(jnp.int32, (G, BE), 0)
    ohfT = ohT.astype(jnp.float32)
    sbT = jnp.where(ohT, sT, NEG)
    bm = jnp.max(sbT, axis=1).reshape(G, 1)
    m_old = m_ref[...]
    m_new = jnp.maximum(m_old, bm)
    scale = jnp.exp(m_old - m_new)
    m_rowT = jax.lax.dot_general(m_new, ohfT, (((0,), (0,)), ((), ())),
                                 preferred_element_type=jnp.float32)
    e_rowT = jnp.exp(sT - m_rowT)
    ewT = ohfT * e_rowT
    z_ref[...] = z_ref[...] * scale + jnp.sum(ewT, axis=1).reshape(G, 1)
    a_ref[...] = a_ref[...] * scale + jax.lax.dot_general(
        ewT, rows, (((1,), (0,)), ((), ())),
        preferred_element_type=jnp.float32)
    m_ref[...] = m_new

    @pl.when(i == pl.num_programs(0) - 1)
    def _():
        gx_ref[...] = a_ref[...] / (z_ref[...] + 1e-16)


def _attn_pool(out, batch3, att_W, att_b):
    return pl.pallas_call(
        _attn_body,
        grid=(NBL,),
        in_specs=[
            pl.BlockSpec((BE, F), lambda i: (i, 0)),
            pl.BlockSpec((1, 1, BE), lambda i: (i, 0, 0)),
            pl.BlockSpec((1, F), lambda i: (0, 0)),
            pl.BlockSpec((1, 1), lambda i: (0, 0)),
        ],
        out_specs=pl.BlockSpec((G, F), lambda i: (0, 0)),
        out_shape=jax.ShapeDtypeStruct((G, F), jnp.float32),
        scratch_shapes=[
            pltpu.VMEM((G, 1), jnp.float32),
            pltpu.VMEM((G, 1), jnp.float32),
            pltpu.VMEM((G, F), jnp.float32),
        ],
    )(out, batch3, att_W, att_b)


def _batchnorm(x, g, b, eps=1e-5):
    m = jnp.mean(x, axis=0)
    v = jnp.var(x, axis=0)
    return (x - m) / jnp.sqrt(v + eps) * g + b


def _prelu(x, a):
    return jnp.where(x >= 0, x, a * x)


def kernel(x, edge_index, edge_attr, line_graph_edge_index, edge_index_batch, params):
    src, dst = edge_index[0], edge_index[1]
    lg0, lg1 = line_graph_edge_index[0], line_graph_edge_index[1]
    batch = edge_index_batch
    eu, ev = _proj(x, params["Wu"], params["Wv"])
    euv = edge_attr @ params["We"].T
    ea = _ea_build(eu, ev, src, dst, euv)

    batch3 = batch.reshape(NBL, 1, BE)
    attW = params["att_W"]
    attb = params["att_b"].reshape(1, 1)

    def _attn_stage(out_n):
        gx = _attn_pool(out_n, batch3, attW, attb)
        return jnp.tanh(gx @ params["Wg"].T + params["bg"])

    out1, bs, bd, cnt = _msg_pass_bin(ea, lg0, lg1, ea)
    outs = [out1]
    gouts = [_attn_stage(out1)]
    for _ in range(T - 1):
        outs.append(_msg_pass_replay(outs[-1], ea, bs, bd, cnt))
        gouts.append(_attn_stage(outs[-1]))
    out_all = jnp.stack(outs, axis=-1)
    gout_all = jnp.stack(gouts, axis=-1)

    scores = jnp.sum(gout_all * params["a"], axis=1, keepdims=True) + params["a_bias"]
    scores = jax.nn.softmax(scores, axis=-1)
    spe = scores[batch]
    o = jnp.sum(out_all * spe, axis=-1)
    x2 = _node_scatter(o, dst, x, jnp.zeros((NROW2L, F), jnp.float32))
    p = params["blk"]
    out1m = _batchnorm(x2, p["bn0_g"], p["bn0_b"]) @ p["W1"].T + p["b1"]
    h = _prelu(_batchnorm(out1m, p["bn2_g"], p["bn2_b"]), p["p3"]) @ p["W4"].T + p["b4"]
    out2 = (h + out1m) / 2.0
    h = _prelu(_batchnorm(out2, p["bn5_g"], p["bn5_b"]), p["p6"]) @ p["W7"].T + p["b7"]
    out3 = (h + out2) / 2.0
    h = _prelu(_batchnorm(out3, p["bn8_g"], p["bn8_b"]), p["p9"]) @ p["W10"].T + p["b10"]
    out4 = (h + out3) / 2.0
    out5 = _prelu(_batchnorm(out4, p["bn11_g"], p["bn11_b"]), p["p12"]) @ p["W13"].T + p["b13"]
    return out5
